# Initial kernel scaffold; baseline (speedup 1.0000x reference)
import jax
import jax.numpy as jnp
from jax.experimental import pallas as pl


def kernel(x, src_uid, src_iid, tgt_iid, W1, b1, W2, W3, b3, W4, b4):
    raise NotImplementedError("write your pallas kernel here")


# speedup vs baseline: 2.4712x; 2.4712x over previous
import functools

import jax
import jax.numpy as jnp
from jax import lax
from jax.experimental import pallas as pl
from jax.experimental.pallas import tpu as pltpu
from jax.experimental.pallas import tpu_sc as plsc

D = 16
TPAD = 208
ROW_BLK = 8192
BLKC = 1024
NW = 32


def _ktable_body(tab_ref, w1_ref, b1_ref, w2t_ref, o_ref):
    rows = tab_ref[...]
    h = jnp.maximum(
        jnp.dot(rows, w1_ref[...], preferred_element_type=jnp.float32)
        + b1_ref[...], 0.0)
    o_ref[...] = jnp.sum(h * w2t_ref[...], axis=1)


def _ktable(src_iid, W1, b1, W2):
    n = src_iid.shape[0]
    nblk = (n + ROW_BLK - 1) // ROW_BLK
    npad = nblk * ROW_BLK
    return pl.pallas_call(
        _ktable_body,
        grid=(nblk,),
        in_specs=[
            pl.BlockSpec((ROW_BLK, D), lambda i: (i, 0)),
            pl.BlockSpec((D, D), lambda i: (0, 0)),
            pl.BlockSpec((1, D), lambda i: (0, 0)),
            pl.BlockSpec((1, D), lambda i: (0, 0)),
        ],
        out_specs=pl.BlockSpec((ROW_BLK,), lambda i: (i,)),
        out_shape=jax.ShapeDtypeStruct((npad,), jnp.float32),
    )(src_iid, W1, b1.reshape(1, D), W2.reshape(1, D))


def _vbroadcast(vec, idxvec):
    dnums = lax.GatherDimensionNumbers(
        offset_dims=(), collapsed_slice_dims=(0,), start_index_map=(0,))
    return lax.gather(vec, idxvec[:, None], dnums, (1,),
                      mode=lax.GatherScatterMode.PROMISE_IN_BOUNDS)


def _sc_body(seqp, uidx, iidx, ktab, siid, suid, tiid,
             his_o, uro, iro,
             idx_a, idx_b, rows_v, k_v, his_v,
             uidx_v, iidx_v, urows_v, irows_v, sem_a, sem_b, bpw):
    c = lax.axis_index("c")
    s = lax.axis_index("s")
    wid = s * 2 + c
    base = wid * bpw
    nch = bpw // 128

    pltpu.sync_copy(uidx.at[pl.ds(base, bpw)], uidx_v)
    pltpu.sync_copy(iidx.at[pl.ds(base, bpw)], iidx_v)
    descs = []
    for j in range(nch):
        descs.append(pltpu.async_copy(
            suid.at[uidx_v.at[pl.ds(j * 128, 128)]],
            urows_v.at[pl.ds(j * 128, 128)], sem_a))
        descs.append(pltpu.async_copy(
            tiid.at[iidx_v.at[pl.ds(j * 128, 128)]],
            irows_v.at[pl.ds(j * 128, 128)], sem_a))
    for d in descs:
        d.wait()
    pltpu.sync_copy(urows_v, uro.at[pl.ds(base, bpw)])
    pltpu.sync_copy(irows_v, iro.at[pl.ds(base, bpw)])

    lane = lax.iota(jnp.int32, 16)
    lane_full = [lax.broadcast(jnp.int32(l), (16,)) for l in range(16)]

    def batch_body(i, carry):
        b = base + i
        off = pl.multiple_of(b * TPAD, 16)
        da = pltpu.async_copy(seqp.at[pl.ds(off, 112)], idx_a, sem_a)
        db = pltpu.async_copy(seqp.at[pl.ds(off + 112, 96)], idx_b, sem_a)
        da.wait()
        db.wait()
        g = [
            pltpu.async_copy(siid.at[idx_a], rows_v.at[pl.ds(0, 112)], sem_b),
            pltpu.async_copy(siid.at[idx_b], rows_v.at[pl.ds(112, 96)], sem_b),
            pltpu.async_copy(ktab.at[idx_a], k_v.at[pl.ds(0, 112)], sem_b),
            pltpu.async_copy(ktab.at[idx_b], k_v.at[pl.ds(112, 96)], sem_b),
        ]
        for d in g:
            d.wait()

        tvs = []
        for v in range(13):
            kv = k_v[pl.ds(v * 16, 16)]
            if v < 7:
                iv = idx_a[pl.ds(v * 16, 16)]
            else:
                iv = idx_b[pl.ds((v - 7) * 16, 16)]
            tv = jnp.where(iv == 0, kv - 1e8, kv)
            if (v + 1) * 16 > 200:
                tv = jnp.where(lane < 200 - v * 16, tv, -1e30)
            tvs.append(tv)
        mv = tvs[0]
        for tv in tvs[1:]:
            mv = jnp.maximum(mv, tv)
        m = jnp.max(mv)
        evs = []
        svec = None
        for v in range(13):
            ev = jnp.exp(tvs[v] - m)
            evs.append(ev)
            svec = ev if svec is None else svec + ev
        rvec = 1.0 / lax.broadcast(jnp.sum(svec), (16,))

        his = jnp.zeros((16,), jnp.float32)
        for v in range(13):
            ev = evs[v]
            for l in range(16):
                w = _vbroadcast(ev, lane_full[l])
                his = his + rows_v[v * 16 + l] * w
        his_v[i] = his * rvec
        return carry

    lax.fori_loop(0, bpw, batch_body, 0)
    pltpu.sync_copy(his_v, his_o.at[pl.ds(base, bpw)])


def _sc_gather(seqp, uidx, iidx, ktab, src_iid, src_uid, tgt_iid, B):
    bpw = B // NW
    mesh = plsc.VectorSubcoreMesh(core_axis_name="c", subcore_axis_name="s")
    f32, i32 = jnp.float32, jnp.int32
    out_type = (
        jax.ShapeDtypeStruct((B, D), f32),
        jax.ShapeDtypeStruct((B, D), f32),
        jax.ShapeDtypeStruct((B, D), f32),
    )
    scratch = [
        pltpu.VMEM((112,), i32),
        pltpu.VMEM((96,), i32),
        pltpu.VMEM((TPAD, D), f32),
        pltpu.VMEM((TPAD,), f32),
        pltpu.VMEM((bpw, D), f32),
        pltpu.VMEM((bpw,), i32),
        pltpu.VMEM((bpw,), i32),
        pltpu.VMEM((bpw, D), f32),
        pltpu.VMEM((bpw, D), f32),
        pltpu.SemaphoreType.DMA,
        pltpu.SemaphoreType.DMA,
    ]
    body = functools.partial(_sc_body, bpw=bpw)
    params = pltpu.CompilerParams(use_tc_tiling_on_sc=False,
                                  needs_layout_passes=False)
    return pl.kernel(body, out_type=out_type, mesh=mesh,
                     compiler_params=params,
                     scratch_types=scratch)(
        seqp, uidx, iidx, ktab, src_iid, src_uid, tgt_iid)


def _final_body(his_ref, ur_ref, ir_ref, w3_ref, b3_ref, w4_ref, b4_ref,
                out_ref, ue_ref, ls_ref):
    his = his_ref[...]
    a = jnp.maximum(
        jnp.dot(his, w3_ref[...], preferred_element_type=jnp.float32)
        + b3_ref[...], 0.0)
    dec = (jnp.dot(a, w4_ref[...], preferred_element_type=jnp.float32)
           + b4_ref[...])
    ur = ur_ref[...]
    ue = ur[:, 0:1] * dec[:, 0:D]
    for k in range(1, D):
        ue = ue + ur[:, k:k + 1] * dec[:, k * D:(k + 1) * D]
    ir = ir_ref[...]
    out_ref[...] = jnp.sum(ue * ir, axis=1, keepdims=True)
    ue_ref[...] = ue
    part = (jnp.sum(ue * ue) + jnp.sum(ir * ir)).reshape(1, 1)

    @pl.when(pl.program_id(0) == 0)
    def _init():
        ls_ref[...] = jnp.zeros_like(ls_ref)

    ls_ref[...] += part


def _final(his, urows, irows, W3, b3, W4, b4, B):
    grid = (B // BLKC,)
    M = W3.shape[1]
    return pl.pallas_call(
        _final_body,
        grid=grid,
        in_specs=[
            pl.BlockSpec((BLKC, D), lambda i: (i, 0)),
            pl.BlockSpec((BLKC, D), lambda i: (i, 0)),
            pl.BlockSpec((BLKC, D), lambda i: (i, 0)),
            pl.BlockSpec((D, M), lambda i: (0, 0)),
            pl.BlockSpec((1, M), lambda i: (0, 0)),
            pl.BlockSpec((M, D * D), lambda i: (0, 0)),
            pl.BlockSpec((1, D * D), lambda i: (0, 0)),
        ],
        out_specs=[
            pl.BlockSpec((BLKC, 1), lambda i: (i, 0)),
            pl.BlockSpec((BLKC, D), lambda i: (i, 0)),
            pl.BlockSpec((1, 1), lambda i: (0, 0)),
        ],
        out_shape=[
            jax.ShapeDtypeStruct((B, 1), jnp.float32),
            jax.ShapeDtypeStruct((B, D), jnp.float32),
            jax.ShapeDtypeStruct((1, 1), jnp.float32),
        ],
    )(his, urows, irows, W3, b3.reshape(1, M), W4, b4.reshape(1, D * D))


def kernel(x, src_uid, src_iid, tgt_iid, W1, b1, W2, W3, b3, W4, b4):
    B = x.shape[0]
    T = x.shape[1] - 2

    seq = x[:, 2:]
    seqp = jnp.pad(seq, ((0, 0), (0, TPAD - T))).reshape(B * TPAD)
    uidx = x[:, 0]
    iidx = x[:, 1]

    ktab = _ktable(src_iid, W1, b1, W2)
    his, urows, irows = _sc_gather(seqp, uidx, iidx, ktab,
                                   src_iid, src_uid, tgt_iid, B)
    out2, ue, ls = _final(his, urows, irows, W3, b3, W4, b4, B)

    output = out2.reshape(B)
    emb = jnp.stack([ue, irows], axis=1)
    emb_loss = jnp.sqrt(ls[0, 0]) / B
    return (output, emb_loss, emb)


# speedup vs baseline: 2.8460x; 1.1517x over previous
import functools

import jax
import jax.numpy as jnp
from jax import lax
from jax.experimental import pallas as pl
from jax.experimental.pallas import tpu as pltpu
from jax.experimental.pallas import tpu_sc as plsc

D = 16
TPAD = 208
ROW_BLK = 8192
BLKC = 1024
NW = 32


def _ktable_body(tab_ref, w1_ref, b1_ref, w2t_ref, o_ref):
    rows = tab_ref[...]
    h = jnp.maximum(
        jnp.dot(rows, w1_ref[...], preferred_element_type=jnp.float32)
        + b1_ref[...], 0.0)
    o_ref[...] = jnp.sum(h * w2t_ref[...], axis=1)


def _ktable(src_iid, W1, b1, W2):
    n = src_iid.shape[0]
    nblk = (n + ROW_BLK - 1) // ROW_BLK
    npad = nblk * ROW_BLK
    return pl.pallas_call(
        _ktable_body,
        grid=(nblk,),
        in_specs=[
            pl.BlockSpec((ROW_BLK, D), lambda i: (i, 0)),
            pl.BlockSpec((D, D), lambda i: (0, 0)),
            pl.BlockSpec((1, D), lambda i: (0, 0)),
            pl.BlockSpec((1, D), lambda i: (0, 0)),
        ],
        out_specs=pl.BlockSpec((ROW_BLK,), lambda i: (i,)),
        out_shape=jax.ShapeDtypeStruct((npad,), jnp.float32),
    )(src_iid, W1, b1.reshape(1, D), W2.reshape(1, D))


def _vbroadcast(vec, idxvec):
    dnums = lax.GatherDimensionNumbers(
        offset_dims=(), collapsed_slice_dims=(0,), start_index_map=(0,))
    return lax.gather(vec, idxvec[:, None], dnums, (1,),
                      mode=lax.GatherScatterMode.PROMISE_IN_BOUNDS)


def _sc_body(seqp, uidx, iidx, ktab, siid, suid, tiid,
             his_o, uro, iro,
             idx_f, rows_v, k_v, his_v,
             uidx_v, iidx_v, urows_v, irows_v,
             sem_u, sem_s0, sem_s1, sem_g0, sem_g1, bpw):
    c = lax.axis_index("c")
    s = lax.axis_index("s")
    wid = s * 2 + c
    base = wid * bpw
    nch = bpw // 128
    sem_s = (sem_s0, sem_s1)
    sem_g = (sem_g0, sem_g1)

    pltpu.sync_copy(uidx.at[pl.ds(base, bpw)], uidx_v)
    pltpu.sync_copy(iidx.at[pl.ds(base, bpw)], iidx_v)
    descs = []
    for j in range(nch):
        descs.append(pltpu.async_copy(
            suid.at[uidx_v.at[pl.ds(j * 128, 128)]],
            urows_v.at[pl.ds(j * 128, 128)], sem_u))
        descs.append(pltpu.async_copy(
            tiid.at[iidx_v.at[pl.ds(j * 128, 128)]],
            irows_v.at[pl.ds(j * 128, 128)], sem_u))
    for d in descs:
        d.wait()
    pltpu.sync_copy(urows_v, uro.at[pl.ds(base, bpw)])
    pltpu.sync_copy(irows_v, iro.at[pl.ds(base, bpw)])

    lane = lax.iota(jnp.int32, 16)
    lane_full = [lax.broadcast(jnp.int32(l), (16,)) for l in range(16)]

    def _seq_start(i, p):
        off = pl.multiple_of((base + i) * TPAD, 16)
        pltpu.async_copy(seqp.at[pl.ds(off, TPAD)], idx_f.at[p], sem_s[p])

    def _seq_wait(p):
        pltpu.make_async_copy(
            seqp.at[pl.ds(0, TPAD)], idx_f.at[p], sem_s[p]).wait()

    def _gather_start(p):
        ia = idx_f.at[p, pl.ds(0, 112)]
        ib = idx_f.at[p, pl.ds(112, 96)]
        pltpu.async_copy(siid.at[ia], rows_v.at[p, pl.ds(0, 112)], sem_g[p])
        pltpu.async_copy(siid.at[ib], rows_v.at[p, pl.ds(112, 96)], sem_g[p])
        pltpu.async_copy(ktab.at[ia], k_v.at[p, pl.ds(0, 112)], sem_g[p])
        pltpu.async_copy(ktab.at[ib], k_v.at[p, pl.ds(112, 96)], sem_g[p])

    def _gather_wait(p):
        ia = idx_f.at[p, pl.ds(0, 112)]
        ib = idx_f.at[p, pl.ds(112, 96)]
        pltpu.make_async_copy(
            siid.at[ia], rows_v.at[p, pl.ds(0, 112)], sem_g[p]).wait()
        pltpu.make_async_copy(
            siid.at[ib], rows_v.at[p, pl.ds(112, 96)], sem_g[p]).wait()
        pltpu.make_async_copy(
            ktab.at[ia], k_v.at[p, pl.ds(0, 112)], sem_g[p]).wait()
        pltpu.make_async_copy(
            ktab.at[ib], k_v.at[p, pl.ds(112, 96)], sem_g[p]).wait()

    def _stage(i, cur):
        nxt = 1 - cur
        _gather_wait(cur)
        @pl.when(i + 1 < bpw)
        def _():
            _seq_wait(nxt)
            _gather_start(nxt)

        ivs = [idx_f[cur, pl.ds(v * 16, 16)] for v in range(13)]

        @pl.when(i + 2 < bpw)
        def _():
            _seq_start(i + 2, cur)

        tvs = []
        for v in range(13):
            kv = k_v[cur, pl.ds(v * 16, 16)]
            tv = jnp.where(ivs[v] == 0, kv - 1e8, kv)
            if (v + 1) * 16 > 200:
                tv = jnp.where(lane < 200 - v * 16, tv, -1e30)
            tvs.append(tv)
        mv = tvs[0]
        for tv in tvs[1:]:
            mv = jnp.maximum(mv, tv)
        m = jnp.max(mv)
        evs = []
        svec = None
        for v in range(13):
            ev = jnp.exp(tvs[v] - m)
            evs.append(ev)
            svec = ev if svec is None else svec + ev
        rvec = 1.0 / lax.broadcast(jnp.sum(svec), (16,))

        his = jnp.zeros((16,), jnp.float32)
        for v in range(13):
            ev = evs[v]
            for l in range(16):
                w = _vbroadcast(ev, lane_full[l])
                his = his + rows_v[cur, v * 16 + l] * w
        his_v[i] = his * rvec

    _seq_start(0, 0)
    _seq_wait(0)
    _gather_start(0)
    _seq_start(1, 1)

    def pair_body(j, carry):
        _stage(2 * j, 0)
        _stage(2 * j + 1, 1)
        return carry

    lax.fori_loop(0, bpw // 2, pair_body, 0)
    pltpu.sync_copy(his_v, his_o.at[pl.ds(base, bpw)])


def _sc_gather(seqp, uidx, iidx, ktab, src_iid, src_uid, tgt_iid, B):
    bpw = B // NW
    mesh = plsc.VectorSubcoreMesh(core_axis_name="c", subcore_axis_name="s")
    f32, i32 = jnp.float32, jnp.int32
    out_type = (
        jax.ShapeDtypeStruct((B, D), f32),
        jax.ShapeDtypeStruct((B, D), f32),
        jax.ShapeDtypeStruct((B, D), f32),
    )
    scratch = [
        pltpu.VMEM((2, TPAD), i32),
        pltpu.VMEM((2, TPAD, D), f32),
        pltpu.VMEM((2, TPAD), f32),
        pltpu.VMEM((bpw, D), f32),
        pltpu.VMEM((bpw,), i32),
        pltpu.VMEM((bpw,), i32),
        pltpu.VMEM((bpw, D), f32),
        pltpu.VMEM((bpw, D), f32),
        pltpu.SemaphoreType.DMA,
        pltpu.SemaphoreType.DMA,
        pltpu.SemaphoreType.DMA,
        pltpu.SemaphoreType.DMA,
        pltpu.SemaphoreType.DMA,
    ]
    body = functools.partial(_sc_body, bpw=bpw)
    params = pltpu.CompilerParams(use_tc_tiling_on_sc=False,
                                  needs_layout_passes=False)
    return pl.kernel(body, out_type=out_type, mesh=mesh,
                     compiler_params=params,
                     scratch_types=scratch)(
        seqp, uidx, iidx, ktab, src_iid, src_uid, tgt_iid)


def _final_body(his_ref, ur_ref, ir_ref, w3_ref, b3_ref, w4_ref, b4_ref,
                out_ref, emb_ref, ls_ref):
    his = his_ref[...]
    a = jnp.maximum(
        jnp.dot(his, w3_ref[...], preferred_element_type=jnp.float32)
        + b3_ref[...], 0.0)
    dec = (jnp.dot(a, w4_ref[...], preferred_element_type=jnp.float32)
           + b4_ref[...])
    ur = ur_ref[...]
    ue = ur[:, 0:1] * dec[:, 0:D]
    for k in range(1, D):
        ue = ue + ur[:, k:k + 1] * dec[:, k * D:(k + 1) * D]
    ir = ir_ref[...]
    out_ref[...] = jnp.sum(ue * ir, axis=1)
    emb_ref[...] = jnp.stack([ue, ir], axis=1)
    part = (jnp.sum(ue * ue) + jnp.sum(ir * ir)).reshape(1, 1)

    @pl.when(pl.program_id(0) == 0)
    def _init():
        ls_ref[...] = jnp.zeros_like(ls_ref)

    ls_ref[...] += part


def _final(his, urows, irows, W3, b3, W4, b4, B):
    grid = (B // BLKC,)
    M = W3.shape[1]
    return pl.pallas_call(
        _final_body,
        grid=grid,
        in_specs=[
            pl.BlockSpec((BLKC, D), lambda i: (i, 0)),
            pl.BlockSpec((BLKC, D), lambda i: (i, 0)),
            pl.BlockSpec((BLKC, D), lambda i: (i, 0)),
            pl.BlockSpec((D, M), lambda i: (0, 0)),
            pl.BlockSpec((1, M), lambda i: (0, 0)),
            pl.BlockSpec((M, D * D), lambda i: (0, 0)),
            pl.BlockSpec((1, D * D), lambda i: (0, 0)),
        ],
        out_specs=[
            pl.BlockSpec((BLKC,), lambda i: (i,)),
            pl.BlockSpec((BLKC, 2, D), lambda i: (i, 0, 0)),
            pl.BlockSpec((1, 1), lambda i: (0, 0)),
        ],
        out_shape=[
            jax.ShapeDtypeStruct((B,), jnp.float32),
            jax.ShapeDtypeStruct((B, 2, D), jnp.float32),
            jax.ShapeDtypeStruct((1, 1), jnp.float32),
        ],
    )(his, urows, irows, W3, b3.reshape(1, M), W4, b4.reshape(1, D * D))


def kernel(x, src_uid, src_iid, tgt_iid, W1, b1, W2, W3, b3, W4, b4):
    B = x.shape[0]
    T = x.shape[1] - 2

    seq = x[:, 2:]
    padv = ((jnp.arange(B, dtype=jnp.int32)[:, None] * (TPAD - T)
             + jnp.arange(TPAD - T, dtype=jnp.int32)[None, :] + 1)
            % jnp.int32(1000000))
    seqp = jnp.concatenate([seq, padv], axis=1).reshape(B * TPAD)
    uidx = x[:, 0]
    iidx = x[:, 1]

    ktab = _ktable(src_iid, W1, b1, W2)
    his, urows, irows = _sc_gather(seqp, uidx, iidx, ktab,
                                   src_iid, src_uid, tgt_iid, B)
    output, emb, ls = _final(his, urows, irows, W3, b3, W4, b4, B)

    emb_loss = jnp.sqrt(ls[0, 0]) / B
    return (output, emb_loss, emb)
